# Initial kernel scaffold; baseline (speedup 1.0000x reference)
import jax
import jax.numpy as jnp
from jax.experimental import pallas as pl


def kernel(seq, graph_len, prompt1, prompt2, prompt3, w_label, w_dff, w_down):
    raise NotImplementedError("write your pallas kernel here")


# speedup vs baseline: 56.4996x; 56.4996x over previous
import functools

import jax
import jax.numpy as jnp
from jax import lax
from jax.experimental import pallas as pl
from jax.experimental.pallas import tpu as pltpu
from jax.experimental.pallas import tpu_sc as plsc

NUM_CORES = 2
NUM_SUBCORES = 16
NUM_WORKERS = NUM_CORES * NUM_SUBCORES
LANES = 16

CH = 64
NCMAX = 8
BUF_ROWS = CH * NCMAX


def _sc_body(N, F, B, SPW, seq_h, meta_h, pvec_h, wsp_h, out_h,
             buf_v, mv_v, pv_v, wv_v, eff_v, row_v, sem):
    nj = F // LANES
    CHW = CH * F

    cid = lax.axis_index("c")
    sid = lax.axis_index("s")
    wid = sid * NUM_CORES + cid

    pltpu.sync_copy(meta_h, mv_v)
    pltpu.sync_copy(pvec_h, pv_v)
    pltpu.sync_copy(wsp_h, wv_v)

    wl0 = wv_v[pl.ds(0, LANES)]
    wl1 = wv_v[pl.ds(LANES, LANES)]
    wl2 = wv_v[pl.ds(2 * LANES, LANES)]
    wd0 = wv_v[pl.ds(3 * LANES, LANES)]
    wd1 = wv_v[pl.ds(4 * LANES, LANES)]
    for j in range(nj):
        p1c = pv_v[pl.ds(j * LANES, LANES)]
        p2c = pv_v[pl.ds(F + j * LANES, LANES)]
        p3c = pv_v[pl.ds(2 * F + j * LANES, LANES)]
        wdc = pv_v[pl.ds(3 * F + j * LANES, LANES)]
        eff_v[pl.ds(j * LANES, LANES)] = (
            wd0 * (1.0 + wl0 * p1c + wl1 * p2c + wl2 * p3c) + wd1 * wdc)

    effs = tuple(eff_v[pl.ds(j * LANES, LANES)] for j in range(nj))
    b0 = wid * SPW

    def seg_body(k, carry):
        b = b0 + k

        @pl.when(b < B)
        def _():
            mv = mv_v[pl.ds(b * 8, LANES)]
            s = mv[0]
            ln = mv[1]
            nc = (ln + (CH - 1)) >> 6

            for c in range(NCMAX):
                @pl.when(c < nc)
                def _(c=c):
                    g = jnp.minimum(s + c * CH, N - CH)
                    pltpu.async_copy(seq_h.at[pl.ds(g * F, CHW)],
                                     buf_v.at[pl.ds(c * CHW, CHW)], sem)
            for c in range(NCMAX):
                @pl.when(c < nc)
                def _(c=c):
                    pltpu.make_async_copy(seq_h.at[pl.ds(0, CHW)],
                                          buf_v.at[pl.ds(c * CHW, CHW)],
                                          sem).wait()

            def chunk_body(c, accs):
                base = s + c * CH
                g = jnp.minimum(base, N - CH)
                d = base - g
                m = jnp.minimum(CH, ln - c * CH)
                lo = c * CH + d

                def row_body(i, accs2):
                    off = i * F
                    new = []
                    for j in range(nj):
                        x = buf_v[pl.ds(off + j * LANES, LANES)]
                        t = effs[j] * x
                        y = jnp.where(t > 0.0, t, jnp.exp(t) - 1.0)
                        new.append(accs2[j] + y)
                    return tuple(new)

                return lax.fori_loop(lo, lo + m, row_body, accs)

            zeros = tuple(jnp.zeros((LANES,), jnp.float32) for _ in range(nj))
            accs = lax.fori_loop(0, nc, chunk_body, zeros)
            for j in range(nj):
                row_v[pl.ds(j * LANES, LANES)] = accs[j]
            pltpu.sync_copy(row_v, out_h.at[pl.ds(b * F, F)])

        return carry

    lax.fori_loop(0, SPW, seg_body, 0)


def kernel(seq, graph_len, prompt1, prompt2, prompt3, w_label, w_dff, w_down):
    N, F = seq.shape
    B = graph_len.shape[0]
    SPW = -(-B // NUM_WORKERS)
    BPAD = SPW * NUM_WORKERS

    offsets = jnp.cumsum(graph_len)
    starts = jnp.minimum(offsets - graph_len, N).astype(jnp.int32)
    ends = jnp.minimum(offsets, N).astype(jnp.int32)
    lens = ends - starts
    meta = jnp.zeros((BPAD * 8 + LANES,), jnp.int32)
    meta = meta.at[0:B * 8:8].set(starts).at[1:B * 8:8].set(lens)

    pvec = jnp.concatenate([prompt1.ravel(), prompt2.ravel(),
                            prompt3.ravel(), w_down.ravel()])
    scalars = jnp.concatenate([w_label.ravel(), w_dff.ravel()])
    wsp = jnp.repeat(scalars, LANES)

    mesh = plsc.VectorSubcoreMesh(core_axis_name="c", subcore_axis_name="s",
                                  num_cores=NUM_CORES,
                                  num_subcores=NUM_SUBCORES)
    body = functools.partial(_sc_body, N, F, B, SPW)
    out_flat = pl.kernel(
        body,
        out_type=jax.ShapeDtypeStruct((B * F,), jnp.float32),
        mesh=mesh,
        scratch_types=[
            pltpu.VMEM((BUF_ROWS * F,), jnp.float32),
            pltpu.VMEM((BPAD * 8 + LANES,), jnp.int32),
            pltpu.VMEM((4 * F,), jnp.float32),
            pltpu.VMEM((5 * LANES,), jnp.float32),
            pltpu.VMEM((F,), jnp.float32),
            pltpu.VMEM((F,), jnp.float32),
            pltpu.SemaphoreType.DMA,
        ],
    )(seq.reshape(-1), meta, pvec, wsp)
    return out_flat.reshape(B, F)


# speedup vs baseline: 69.1401x; 1.2237x over previous
import functools

import jax
import jax.numpy as jnp
from jax import lax
from jax.experimental import pallas as pl
from jax.experimental.pallas import tpu as pltpu
from jax.experimental.pallas import tpu_sc as plsc

NUM_CORES = 2
NUM_SUBCORES = 16
NUM_WORKERS = NUM_CORES * NUM_SUBCORES
LANES = 16

CH = 56
NCMAX = 9
SEGROWS = CH * NCMAX


def _sc_body(N, F, B, seq_h, meta_h, wlh_h, pvec_h, wsp_h, out_h,
             buf_v, m0_v, m1_v, wl_v, pv_v, wv_v, eff_v, row_v,
             sem_a, sem_b):
    nj = F // LANES
    CHW = CH * F
    BASE1 = SEGROWS * F

    cid = lax.axis_index("c")
    sid = lax.axis_index("s")
    wid = sid * NUM_CORES + cid

    pltpu.sync_copy(pvec_h, pv_v)
    pltpu.sync_copy(wsp_h, wv_v)
    pltpu.sync_copy(wlh_h.at[pl.ds(wid * 16, LANES)], wl_v)

    wl0 = wv_v[pl.ds(0, LANES)]
    wl1 = wv_v[pl.ds(LANES, LANES)]
    wl2 = wv_v[pl.ds(2 * LANES, LANES)]
    wd0 = wv_v[pl.ds(3 * LANES, LANES)]
    wd1 = wv_v[pl.ds(4 * LANES, LANES)]
    for j in range(nj):
        p1c = pv_v[pl.ds(j * LANES, LANES)]
        p2c = pv_v[pl.ds(F + j * LANES, LANES)]
        p3c = pv_v[pl.ds(2 * F + j * LANES, LANES)]
        wdc = pv_v[pl.ds(3 * F + j * LANES, LANES)]
        eff_v[pl.ds(j * LANES, LANES)] = (
            wd0 * (1.0 + wl0 * p1c + wl1 * p2c + wl2 * p3c) + wd1 * wdc)

    effs = tuple(eff_v[pl.ds(j * LANES, LANES)] for j in range(nj))

    wlv = wl_v[pl.ds(0, LANES)]
    lo = wlv[0]
    hi = wlv[1]

    def read_meta(k, mb):
        pltpu.sync_copy(meta_h.at[pl.ds(k * 16, LANES)], mb)

    def seg_params(mb):
        mv = mb[pl.ds(0, LANES)]
        return mv[0], mv[1]

    def fire(mb, base, sem):
        s, ln = seg_params(mb)
        nc = (ln + (CH - 1)) // CH
        for c in range(NCMAX):
            @pl.when(c < nc)
            def _(c=c):
                g = jnp.minimum(s + c * CH, N - CH)
                pltpu.async_copy(seq_h.at[pl.ds(g * F, CHW)],
                                 buf_v.at[pl.ds(base + c * CHW, CHW)], sem)

    def drain_compute_write(mb, base, sem, b):
        s, ln = seg_params(mb)
        nc = (ln + (CH - 1)) // CH
        for c in range(NCMAX):
            @pl.when(c < nc)
            def _(c=c):
                pltpu.make_async_copy(
                    seq_h.at[pl.ds(0, CHW)],
                    buf_v.at[pl.ds(base + c * CHW, CHW)], sem).wait()

        def chunk_body(c, accs):
            cbase = s + c * CH
            g = jnp.minimum(cbase, N - CH)
            d = cbase - g
            m = jnp.minimum(CH, ln - c * CH)
            rlo = (base // F) + c * CH + d

            def row_body(i, accs2):
                off = i * F
                new = []
                for j in range(nj):
                    x = buf_v[pl.ds(off + j * LANES, LANES)]
                    t = effs[j] * x
                    y = jnp.where(t > 0.0, t, jnp.exp(t) - 1.0)
                    new.append(accs2[j] + y)
                return tuple(new)

            return lax.fori_loop(rlo, rlo + m, row_body, accs)

        zeros = tuple(jnp.zeros((LANES,), jnp.float32) for _ in range(nj))
        accs = lax.fori_loop(0, nc, chunk_body, zeros)
        for j in range(nj):
            row_v[pl.ds(j * LANES, LANES)] = accs[j]
        pltpu.sync_copy(row_v, out_h.at[pl.ds(b * F, F)])

    @pl.when(lo < hi)
    def _():
        read_meta(lo, m0_v)
        fire(m0_v, 0, sem_a)

    npairs = (hi - lo + 1) >> 1

    def pair_body(kk, carry):
        k0 = lo + 2 * kk
        k1 = k0 + 1

        @pl.when(k1 < hi)
        def _():
            read_meta(k1, m1_v)
            fire(m1_v, BASE1, sem_b)

        drain_compute_write(m0_v, 0, sem_a, k0)

        @pl.when(k0 + 2 < hi)
        def _():
            read_meta(k0 + 2, m0_v)
            fire(m0_v, 0, sem_a)

        @pl.when(k1 < hi)
        def _():
            drain_compute_write(m1_v, BASE1, sem_b, k1)

        return carry

    lax.fori_loop(0, npairs, pair_body, 0)


def kernel(seq, graph_len, prompt1, prompt2, prompt3, w_label, w_dff, w_down):
    N, F = seq.shape
    B = graph_len.shape[0]

    offsets = jnp.cumsum(graph_len)
    starts = jnp.minimum(offsets - graph_len, N).astype(jnp.int32)
    ends = jnp.minimum(offsets, N).astype(jnp.int32)
    lens = ends - starts
    cum = jnp.cumsum(lens)
    totalr = cum[B - 1]
    targets = (jnp.arange(1, NUM_WORKERS, dtype=jnp.int32) * totalr) // NUM_WORKERS
    mids = jnp.searchsorted(cum, targets, side="left").astype(jnp.int32)
    wb = jnp.concatenate([jnp.zeros((1,), jnp.int32), mids,
                          jnp.full((1,), B, jnp.int32)])

    meta = jnp.zeros((B * 16 + LANES,), jnp.int32)
    meta = meta.at[0:B * 16:16].set(starts).at[1:B * 16:16].set(lens)
    wlh = jnp.zeros((NUM_WORKERS * 16 + LANES,), jnp.int32)
    idx = jnp.arange(NUM_WORKERS) * 16
    wlh = wlh.at[idx].set(wb[:NUM_WORKERS]).at[idx + 1].set(wb[1:])

    pvec = jnp.concatenate([prompt1.ravel(), prompt2.ravel(),
                            prompt3.ravel(), w_down.ravel()])
    scalars = jnp.concatenate([w_label.ravel(), w_dff.ravel()])
    wsp = jnp.repeat(scalars, LANES)

    mesh = plsc.VectorSubcoreMesh(core_axis_name="c", subcore_axis_name="s",
                                  num_cores=NUM_CORES,
                                  num_subcores=NUM_SUBCORES)
    body = functools.partial(_sc_body, N, F, B)
    out_flat = pl.kernel(
        body,
        out_type=jax.ShapeDtypeStruct((B * F,), jnp.float32),
        mesh=mesh,
        scratch_types=[
            pltpu.VMEM((2 * SEGROWS * F,), jnp.float32),
            pltpu.VMEM((LANES,), jnp.int32),
            pltpu.VMEM((LANES,), jnp.int32),
            pltpu.VMEM((LANES,), jnp.int32),
            pltpu.VMEM((4 * F,), jnp.float32),
            pltpu.VMEM((5 * LANES,), jnp.float32),
            pltpu.VMEM((F,), jnp.float32),
            pltpu.VMEM((F,), jnp.float32),
            pltpu.SemaphoreType.DMA,
            pltpu.SemaphoreType.DMA,
        ],
    )(seq.reshape(-1), meta, wlh, pvec, wsp)
    return out_flat.reshape(B, F)
